# baseline (speedup 1.0000x reference)
import functools

import jax
import jax.numpy as jnp
from jax import lax
from jax.experimental import pallas as pl
from jax.experimental.pallas import tpu as pltpu
from jax.experimental.pallas import tpu_sc as plsc

N = 10000
E = 320000
D = 128
NC = 2
NS = 16
NP = 10240
RPT = NP // NS

EC = E // NC
ET = EC // NS
CS = 128
FCH, FTAIL = ET // CS, ET % CS
GT = E // (NC * NS)
GCS = 80
GCH = 2 * (GT // GCS)
GTAIL = GT % GCS
YRT = 624

_mesh = plsc.VectorSubcoreMesh(core_axis_name="c", subcore_axis_name="s")

_IDX = lambda n: [pltpu.VMEM((128,), jnp.int32) for _ in range(n)]
_SEM = lambda n: [pltpu.SemaphoreType.DMA for _ in range(n)]
_ROWS = lambda n: [pltpu.VMEM((128, D), jnp.float32) for _ in range(n)]


def _fill_ones(ref, n):
    @pl.loop(0, n // 16)
    def _(i):
        ref[pl.ds(i * 16, 16)] = jnp.ones((16,), jnp.float32)


def _agg_pipeline(feat_hbm, src_hbm, dst_hbm, agg_sh, deg_sh, sidx, didx,
                  rows, ones_v, isem, gsem, ssem, dgsem, sidx_t, didx_t,
                  rows_t, ones_t, f_base, with_deg):
    H = CS // 2

    def idx_issue(q, b4):
        off = pl.multiple_of(f_base + q * CS, 8)
        pltpu.async_copy(src_hbm.at[pl.ds(off, CS)], sidx[b4], isem[b4])
        pltpu.async_copy(dst_hbm.at[pl.ds(off, CS)], didx[b4], isem[b4])

    def idx_wait(b4):
        pltpu.make_async_copy(src_hbm.at[pl.ds(0, CS)], sidx[b4], isem[b4]).wait()
        pltpu.make_async_copy(dst_hbm.at[pl.ds(0, CS)], didx[b4], isem[b4]).wait()

    def gathers_issue(b4, w):
        pltpu.async_copy(feat_hbm.at[sidx[b4].at[pl.ds(0, H)]],
                         rows.at[pl.ds(w * CS, H)], gsem[w])
        pltpu.async_copy(feat_hbm.at[sidx[b4].at[pl.ds(H, H)]],
                         rows.at[pl.ds(w * CS + H, H)], gsem[w])

    def gathers_wait(b4, w):
        pltpu.make_async_copy(feat_hbm.at[sidx[b4].at[pl.ds(0, H)]],
                              rows.at[pl.ds(w * CS, H)], gsem[w]).wait()
        pltpu.make_async_copy(feat_hbm.at[sidx[b4].at[pl.ds(H, H)]],
                              rows.at[pl.ds(w * CS + H, H)], gsem[w]).wait()

    def scat_issue(b4, w):
        pltpu.async_copy(rows.at[pl.ds(w * CS, CS)], agg_sh.at[didx[b4]],
                         ssem[w], add=True)
        if with_deg:
            pltpu.async_copy(ones_v, deg_sh.at[didx[b4]], dgsem[w], add=True)

    def scat_wait(b4, w):
        pltpu.make_async_copy(rows.at[pl.ds(w * CS, CS)], agg_sh.at[didx[b4]],
                              ssem[w]).wait()
        if with_deg:
            pltpu.make_async_copy(ones_v, deg_sh.at[didx[b4]], dgsem[w]).wait()

    def B(q, b4):
        traced = not isinstance(q, int)
        w = b4 % 2
        if traced or q >= 2:
            scat_wait((b4 + 2) % 4, w)
        idx_wait(b4)
        gathers_issue(b4, w)
        if traced or q >= 1:
            jp4 = (b4 + 3) % 4
            gathers_wait(jp4, 1 - w)
            scat_issue(jp4, 1 - w)
        if traced or q + 2 < FCH:
            idx_issue(q + 2, (b4 + 2) % 4)

    idx_issue(0, 0)
    idx_issue(1, 1)
    for q in range(4):
        B(q, q % 4)

    @pl.loop(4, FCH - 2, step=4)
    def _(v):
        for u in range(4):
            B(v + u, u)

    for q in range(FCH - 2, FCH):
        B(q, q % 4)

    b4l, wl = (FCH - 1) % 4, (FCH - 1) % 2
    gathers_wait(b4l, wl)
    scat_issue(b4l, wl)
    scat_wait((FCH - 2) % 4, (FCH - 2) % 2)
    scat_wait(b4l, wl)

    off = f_base + FCH * CS
    pltpu.sync_copy(src_hbm.at[pl.ds(off, FTAIL)], sidx_t)
    pltpu.sync_copy(dst_hbm.at[pl.ds(off, FTAIL)], didx_t)
    pltpu.async_copy(feat_hbm.at[sidx_t], rows_t, gsem[0]).wait()
    pltpu.sync_copy(rows_t, agg_sh.at[didx_t], add=True)
    if with_deg:
        pltpu.sync_copy(ones_t, deg_sh.at[didx_t], add=True)


def _sc_agg(feat, src, dst, z2, z1, with_deg):
    out_type = [jax.ShapeDtypeStruct((2 * NP, D), jnp.float32)]
    if with_deg:
        out_type.append(jax.ShapeDtypeStruct((2 * NP,), jnp.float32))

    @functools.partial(
        pl.kernel,
        out_type=tuple(out_type),
        mesh=_mesh,
        scratch_types=dict(
            agg_sh=pltpu.VMEM_SHARED((NP, D), jnp.float32),
            deg_sh=pltpu.VMEM_SHARED((NP,), jnp.float32),
            sidx=[pltpu.VMEM((CS,), jnp.int32) for _ in range(4)],
            didx=[pltpu.VMEM((CS,), jnp.int32) for _ in range(4)],
            sidx_t=pltpu.VMEM((FTAIL,), jnp.int32),
            didx_t=pltpu.VMEM((FTAIL,), jnp.int32),
            ones_v=pltpu.VMEM((CS,), jnp.float32),
            ones_t=pltpu.VMEM((FTAIL,), jnp.float32),
            rows=pltpu.VMEM((2 * CS, D), jnp.float32),
            rows_t=pltpu.VMEM((FTAIL, D), jnp.float32),
            isem=_SEM(4), gsem=_SEM(2), ssem=_SEM(2), dgsem=_SEM(2),
        ),
    )
    def k(feat_hbm, src_hbm, dst_hbm, z2_hbm, z1_hbm, *out_and_scratch,
          agg_sh, deg_sh, sidx, didx, sidx_t, didx_t, ones_v, ones_t,
          rows, rows_t, isem, gsem, ssem, dgsem):
        if with_deg:
            part_hbm, degp_hbm = out_and_scratch
        else:
            (part_hbm,) = out_and_scratch
            degp_hbm = None
        c = lax.axis_index("c")
        s = lax.axis_index("s")
        r0 = s * RPT

        pltpu.sync_copy(z2_hbm.at[pl.ds(r0, RPT)], agg_sh.at[pl.ds(r0, RPT)])
        if with_deg:
            pltpu.sync_copy(z1_hbm.at[pl.ds(r0, RPT)], deg_sh.at[pl.ds(r0, RPT)])
            _fill_ones(ones_v, CS)
            _fill_ones(ones_t, FTAIL)
        plsc.subcore_barrier()

        _agg_pipeline(feat_hbm, src_hbm, dst_hbm, agg_sh, deg_sh, sidx, didx,
                      rows, ones_v, isem, gsem, ssem, dgsem, sidx_t, didx_t,
                      rows_t, ones_t, c * EC + s * ET, with_deg)
        plsc.subcore_barrier()

        pltpu.sync_copy(agg_sh.at[pl.ds(r0, RPT)],
                        part_hbm.at[pl.ds(c * NP + r0, RPT)])
        if with_deg:
            pltpu.sync_copy(deg_sh.at[pl.ds(r0, RPT)],
                            degp_hbm.at[pl.ds(c * NP + r0, RPT)])

    return k(feat, src, dst, z2, z1)


def _sc_gather_out(y, src, dst):
    @functools.partial(
        pl.kernel,
        out_type=(
            jax.ShapeDtypeStruct((E, D), jnp.float32),
            jax.ShapeDtypeStruct((E, D), jnp.float32),
        ),
        mesh=_mesh,
        scratch_types=dict(
            y_sh=pltpu.VMEM_SHARED((N, D), jnp.float32),
            idx=[pltpu.VMEM((GCS,), jnp.int32) for _ in range(4)],
            rows=[pltpu.VMEM((GCS, D), jnp.float32) for _ in range(4)],
            isem=_SEM(4), gsem=_SEM(4), wsem=_SEM(4),
        ),
    )
    def k(y_hbm, src_hbm, dst_hbm, sf_hbm, df_hbm, *,
          y_sh, idx, rows, isem, gsem, wsem):
        c = lax.axis_index("c")
        s = lax.axis_index("s")
        base = (c * NS + s) * GT

        yr = s * YRT
        pltpu.sync_copy(y_hbm.at[pl.ds(yr, YRT)], y_sh.at[pl.ds(yr, YRT)])

        @pl.when(s == NS - 1)
        def _():
            pltpu.sync_copy(y_hbm.at[pl.ds(NS * YRT, N - NS * YRT)],
                            y_sh.at[pl.ds(NS * YRT, N - NS * YRT)])

        plsc.subcore_barrier()

        def off_of(jj):
            return pl.multiple_of(base + (jj // 2) * GCS, 8)

        def idx_issue(jj, b4, even):
            ref = src_hbm if even else dst_hbm
            pltpu.async_copy(ref.at[pl.ds(off_of(jj), GCS)], idx[b4], isem[b4])

        def idx_wait(b4):
            pltpu.make_async_copy(src_hbm.at[pl.ds(0, GCS)], idx[b4],
                                  isem[b4]).wait()

        def write_issue(jj, b4, even):
            out = sf_hbm if even else df_hbm
            pltpu.async_copy(rows[b4], out.at[pl.ds(off_of(jj), GCS)], wsem[b4])

        def write_wait(b4, even):
            out = sf_hbm if even else df_hbm
            pltpu.make_async_copy(rows[b4], out.at[pl.ds(0, GCS)],
                                  wsem[b4]).wait()

        def B(jj, b4, even):
            if not isinstance(jj, int) or jj >= 4:
                write_wait(b4, even)
            idx_wait(b4)
            pltpu.async_copy(y_sh.at[idx[b4]], rows[b4], gsem[b4])
            if not isinstance(jj, int) or jj >= 2:
                jp4 = (b4 + 2) % 4
                pltpu.make_async_copy(y_sh.at[idx[jp4]], rows[jp4],
                                      gsem[jp4]).wait()
                write_issue(jj - 2, jp4, even)
            if not isinstance(jj, int):
                idx_issue(jj + 2, (b4 + 2) % 4, even)
            elif jj + 2 < GCH:
                idx_issue(jj + 2, (b4 + 2) % 4, even)

        idx_issue(0, 0, True)
        idx_issue(1, 1, False)
        for jj in range(4):
            B(jj, jj % 4, jj % 2 == 0)

        @pl.loop(4, GCH - 6, step=4)
        def _(v):
            for u in range(4):
                B(v + u, u, u % 2 == 0)

        for jj in range(GCH - 6, GCH):
            B(jj, jj % 4, jj % 2 == 0)

        for jj in (GCH - 2, GCH - 1):
            b4, even = jj % 4, jj % 2 == 0
            pltpu.make_async_copy(y_sh.at[idx[b4]], rows[b4], gsem[b4]).wait()
            write_issue(jj, b4, even)
        for jj in range(GCH - 4, GCH):
            write_wait(jj % 4, jj % 2 == 0)

    return k(y, src, dst)


def _tc_dense(x, part, degp, W_s, W_n, b, relu):
    R = 1000
    part3 = part.reshape(2, NP, D)
    deg3 = degp.reshape(2, NP, 1)
    b2d = b.reshape(1, D)

    def body(x_ref, p0_ref, p1_ref, d_ref, ws_ref, wn_ref, b_ref, o_ref):
        dsum = d_ref[0] + d_ref[1]
        scale = 1.0 / jnp.maximum(dsum, 1.0)
        agg = (p0_ref[0] + p1_ref[0]) * scale
        acc = jnp.dot(x_ref[...], ws_ref[...], preferred_element_type=jnp.float32)
        acc = acc + jnp.dot(agg, wn_ref[...], preferred_element_type=jnp.float32)
        acc = acc + b_ref[...]
        if relu:
            acc = jnp.maximum(acc, 0.0)
        o_ref[...] = acc

    return pl.pallas_call(
        body,
        grid=(N // R,),
        in_specs=[
            pl.BlockSpec((R, D), lambda i: (i, 0)),
            pl.BlockSpec((1, R, D), lambda i: (0, i, 0)),
            pl.BlockSpec((1, R, D), lambda i: (1, i, 0)),
            pl.BlockSpec((2, R, 1), lambda i: (0, i, 0)),
            pl.BlockSpec((D, D), lambda i: (0, 0)),
            pl.BlockSpec((D, D), lambda i: (0, 0)),
            pl.BlockSpec((1, D), lambda i: (0, 0)),
        ],
        out_specs=pl.BlockSpec((R, D), lambda i: (i, 0)),
        out_shape=jax.ShapeDtypeStruct((N, D), jnp.float32),
    )(x, part3, part3, deg3, W_s, W_n, b2d)


def kernel(x, edge_index, W_self1, W_neigh1, b1, W_self2, W_neigh2, b2):
    src = edge_index[0].astype(jnp.int32)
    dst = edge_index[1].astype(jnp.int32)
    z2 = jnp.zeros((NP, D), jnp.float32)
    z1 = jnp.zeros((NP,), jnp.float32)

    part1, degp = _sc_agg(x, src, dst, z2, z1, with_deg=True)
    h = _tc_dense(x, part1, degp, W_self1, W_neigh1, b1, relu=True)
    (part2,) = _sc_agg(h, src, dst, z2, z1, with_deg=False)
    out2 = _tc_dense(h, part2, degp, W_self2, W_neigh2, b2, relu=False)
    src_feat, dst_feat = _sc_gather_out(out2, src, dst)
    return (src_feat, dst_feat)


# speedup vs baseline: 1.0316x; 1.0316x over previous
import functools

import jax
import jax.numpy as jnp
from jax import lax
from jax.experimental import pallas as pl
from jax.experimental.pallas import tpu as pltpu
from jax.experimental.pallas import tpu_sc as plsc

N = 10000
E = 320000
D = 128
NC = 2
NS = 16
NP = 10240
RPT = NP // NS

EC = E // NC
ET = EC // NS
CS = 64
FCH, FTAIL = ET // CS, ET % CS
GT = E // (NC * NS)
GCS = 80
GCH = 2 * (GT // GCS)
GTAIL = GT % GCS
YRT = 624

_mesh = plsc.VectorSubcoreMesh(core_axis_name="c", subcore_axis_name="s")

_IDX = lambda n: [pltpu.VMEM((128,), jnp.int32) for _ in range(n)]
_SEM = lambda n: [pltpu.SemaphoreType.DMA for _ in range(n)]
_ROWS = lambda n: [pltpu.VMEM((128, D), jnp.float32) for _ in range(n)]


def _fill_ones(ref, n):
    @pl.loop(0, n // 16)
    def _(i):
        ref[pl.ds(i * 16, 16)] = jnp.ones((16,), jnp.float32)


def _agg_pipeline(feat_hbm, src_hbm, dst_hbm, agg_sh, deg_sh, sidx, didx,
                  rows, ones_v, isem, gsem, ssem, dgsem, sidx_t, didx_t,
                  rows_t, ones_t, f_base, with_deg):
    def idx_issue(jj, b4, b8):
        off = pl.multiple_of(f_base + jj * CS, 8)
        pltpu.async_copy(src_hbm.at[pl.ds(off, CS)], sidx[b4], isem[b4])
        pltpu.async_copy(dst_hbm.at[pl.ds(off, CS)], didx[b8], isem[b4])

    def idx_wait(b4, b8):
        pltpu.make_async_copy(src_hbm.at[pl.ds(0, CS)], sidx[b4], isem[b4]).wait()
        pltpu.make_async_copy(dst_hbm.at[pl.ds(0, CS)], didx[b8], isem[b4]).wait()

    def scat_issue(b8, b4):
        pltpu.async_copy(rows[b4], agg_sh.at[didx[b8]], ssem[b4], add=True)
        if with_deg:
            pltpu.async_copy(ones_v, deg_sh.at[didx[b8]], dgsem[b4], add=True)

    def scat_wait(b8, b4):
        pltpu.make_async_copy(rows[b4], agg_sh.at[didx[b8]], ssem[b4]).wait()
        if with_deg:
            pltpu.make_async_copy(ones_v, deg_sh.at[didx[b8]], dgsem[b4]).wait()

    def gather_wait(b4):
        pltpu.make_async_copy(feat_hbm.at[sidx[b4]], rows[b4], gsem[b4]).wait()

    def B(jj, b4, b8):
        traced = not isinstance(jj, int)
        if traced or jj >= 3:
            scat_wait((b8 + 5) % 8, (b4 + 1) % 4)
        idx_wait(b4, b8)
        pltpu.async_copy(feat_hbm.at[sidx[b4]], rows[b4], gsem[b4])
        if traced or jj >= 2:
            jp4, jp8 = (b4 + 2) % 4, (b8 + 6) % 8
            gather_wait(jp4)
            scat_issue(jp8, jp4)
        if traced or jj + 2 < FCH:
            idx_issue(jj + 2, (b4 + 2) % 4, (b8 + 2) % 8)

    idx_issue(0, 0, 0)
    idx_issue(1, 1, 1)
    for jj in range(4):
        B(jj, jj % 4, jj % 8)

    @pl.loop(4, FCH - 8, step=8)
    def _(v):
        for u in range(8):
            B(v + u, (4 + u) % 4, (4 + u) % 8)

    for jj in range(FCH - 8, FCH):
        B(jj, jj % 4, jj % 8)

    scat_wait((FCH - 3) % 8, (FCH - 3) % 4)
    for jj in (FCH - 2, FCH - 1):
        gather_wait(jj % 4)
        scat_issue(jj % 8, jj % 4)
        scat_wait(jj % 8, jj % 4)

    off = f_base + FCH * CS
    pltpu.sync_copy(src_hbm.at[pl.ds(off, FTAIL)], sidx_t)
    pltpu.sync_copy(dst_hbm.at[pl.ds(off, FTAIL)], didx_t)
    pltpu.async_copy(feat_hbm.at[sidx_t], rows_t, gsem[0]).wait()
    pltpu.sync_copy(rows_t, agg_sh.at[didx_t], add=True)
    if with_deg:
        pltpu.sync_copy(ones_t, deg_sh.at[didx_t], add=True)


def _sc_agg(feat, src, dst, z2, z1, with_deg):
    out_type = [jax.ShapeDtypeStruct((2 * NP, D), jnp.float32)]
    if with_deg:
        out_type.append(jax.ShapeDtypeStruct((2 * NP,), jnp.float32))

    @functools.partial(
        pl.kernel,
        out_type=tuple(out_type),
        mesh=_mesh,
        scratch_types=dict(
            agg_sh=pltpu.VMEM_SHARED((NP, D), jnp.float32),
            deg_sh=pltpu.VMEM_SHARED((NP,), jnp.float32),
            sidx=[pltpu.VMEM((CS,), jnp.int32) for _ in range(4)],
            didx=[pltpu.VMEM((CS,), jnp.int32) for _ in range(8)],
            sidx_t=pltpu.VMEM((FTAIL,), jnp.int32),
            didx_t=pltpu.VMEM((FTAIL,), jnp.int32),
            ones_v=pltpu.VMEM((CS,), jnp.float32),
            ones_t=pltpu.VMEM((FTAIL,), jnp.float32),
            rows=[pltpu.VMEM((CS, D), jnp.float32) for _ in range(4)],
            rows_t=pltpu.VMEM((FTAIL, D), jnp.float32),
            isem=_SEM(4), gsem=_SEM(4), ssem=_SEM(4), dgsem=_SEM(4),
        ),
    )
    def k(feat_hbm, src_hbm, dst_hbm, z2_hbm, z1_hbm, *out_and_scratch,
          agg_sh, deg_sh, sidx, didx, sidx_t, didx_t, ones_v, ones_t,
          rows, rows_t, isem, gsem, ssem, dgsem):
        if with_deg:
            part_hbm, degp_hbm = out_and_scratch
        else:
            (part_hbm,) = out_and_scratch
            degp_hbm = None
        c = lax.axis_index("c")
        s = lax.axis_index("s")
        r0 = s * RPT

        pltpu.sync_copy(z2_hbm.at[pl.ds(r0, RPT)], agg_sh.at[pl.ds(r0, RPT)])
        if with_deg:
            pltpu.sync_copy(z1_hbm.at[pl.ds(r0, RPT)], deg_sh.at[pl.ds(r0, RPT)])
            _fill_ones(ones_v, CS)
            _fill_ones(ones_t, FTAIL)
        plsc.subcore_barrier()

        _agg_pipeline(feat_hbm, src_hbm, dst_hbm, agg_sh, deg_sh, sidx, didx,
                      rows, ones_v, isem, gsem, ssem, dgsem, sidx_t, didx_t,
                      rows_t, ones_t, c * EC + s * ET, with_deg)
        plsc.subcore_barrier()

        pltpu.sync_copy(agg_sh.at[pl.ds(r0, RPT)],
                        part_hbm.at[pl.ds(c * NP + r0, RPT)])
        if with_deg:
            pltpu.sync_copy(deg_sh.at[pl.ds(r0, RPT)],
                            degp_hbm.at[pl.ds(c * NP + r0, RPT)])

    return k(feat, src, dst, z2, z1)


def _sc_gather_out(y, src, dst):
    @functools.partial(
        pl.kernel,
        out_type=(
            jax.ShapeDtypeStruct((E, D), jnp.float32),
            jax.ShapeDtypeStruct((E, D), jnp.float32),
        ),
        mesh=_mesh,
        scratch_types=dict(
            y_sh=pltpu.VMEM_SHARED((N, D), jnp.float32),
            idx=[pltpu.VMEM((GCS,), jnp.int32) for _ in range(4)],
            rows=[pltpu.VMEM((GCS, D), jnp.float32) for _ in range(4)],
            isem=_SEM(4), gsem=_SEM(4), wsem=_SEM(4),
        ),
    )
    def k(y_hbm, src_hbm, dst_hbm, sf_hbm, df_hbm, *,
          y_sh, idx, rows, isem, gsem, wsem):
        c = lax.axis_index("c")
        s = lax.axis_index("s")
        base = (c * NS + s) * GT

        yr = s * YRT
        pltpu.sync_copy(y_hbm.at[pl.ds(yr, YRT)], y_sh.at[pl.ds(yr, YRT)])

        @pl.when(s == NS - 1)
        def _():
            pltpu.sync_copy(y_hbm.at[pl.ds(NS * YRT, N - NS * YRT)],
                            y_sh.at[pl.ds(NS * YRT, N - NS * YRT)])

        plsc.subcore_barrier()

        def off_of(jj):
            return pl.multiple_of(base + (jj // 2) * GCS, 8)

        def idx_issue(jj, b4, even):
            ref = src_hbm if even else dst_hbm
            pltpu.async_copy(ref.at[pl.ds(off_of(jj), GCS)], idx[b4], isem[b4])

        def idx_wait(b4):
            pltpu.make_async_copy(src_hbm.at[pl.ds(0, GCS)], idx[b4],
                                  isem[b4]).wait()

        def write_issue(jj, b4, even):
            out = sf_hbm if even else df_hbm
            pltpu.async_copy(rows[b4], out.at[pl.ds(off_of(jj), GCS)], wsem[b4])

        def write_wait(b4, even):
            out = sf_hbm if even else df_hbm
            pltpu.make_async_copy(rows[b4], out.at[pl.ds(0, GCS)],
                                  wsem[b4]).wait()

        def B(jj, b4, even):
            if not isinstance(jj, int) or jj >= 4:
                write_wait(b4, even)
            idx_wait(b4)
            pltpu.async_copy(y_sh.at[idx[b4]], rows[b4], gsem[b4])
            if not isinstance(jj, int) or jj >= 2:
                jp4 = (b4 + 2) % 4
                pltpu.make_async_copy(y_sh.at[idx[jp4]], rows[jp4],
                                      gsem[jp4]).wait()
                write_issue(jj - 2, jp4, even)
            if not isinstance(jj, int):
                idx_issue(jj + 2, (b4 + 2) % 4, even)
            elif jj + 2 < GCH:
                idx_issue(jj + 2, (b4 + 2) % 4, even)

        idx_issue(0, 0, True)
        idx_issue(1, 1, False)
        for jj in range(4):
            B(jj, jj % 4, jj % 2 == 0)

        @pl.loop(4, GCH - 6, step=4)
        def _(v):
            for u in range(4):
                B(v + u, u, u % 2 == 0)

        for jj in range(GCH - 6, GCH):
            B(jj, jj % 4, jj % 2 == 0)

        for jj in (GCH - 2, GCH - 1):
            b4, even = jj % 4, jj % 2 == 0
            pltpu.make_async_copy(y_sh.at[idx[b4]], rows[b4], gsem[b4]).wait()
            write_issue(jj, b4, even)
        for jj in range(GCH - 4, GCH):
            write_wait(jj % 4, jj % 2 == 0)

    return k(y, src, dst)


def _tc_dense(x, part, degp, W_s, W_n, b, relu):
    R = 1000
    part3 = part.reshape(2, NP, D)
    deg3 = degp.reshape(2, NP, 1)
    b2d = b.reshape(1, D)

    def body(x_ref, p0_ref, p1_ref, d_ref, ws_ref, wn_ref, b_ref, o_ref):
        dsum = d_ref[0] + d_ref[1]
        scale = 1.0 / jnp.maximum(dsum, 1.0)
        agg = (p0_ref[0] + p1_ref[0]) * scale
        acc = jnp.dot(x_ref[...], ws_ref[...], preferred_element_type=jnp.float32)
        acc = acc + jnp.dot(agg, wn_ref[...], preferred_element_type=jnp.float32)
        acc = acc + b_ref[...]
        if relu:
            acc = jnp.maximum(acc, 0.0)
        o_ref[...] = acc

    return pl.pallas_call(
        body,
        grid=(N // R,),
        in_specs=[
            pl.BlockSpec((R, D), lambda i: (i, 0)),
            pl.BlockSpec((1, R, D), lambda i: (0, i, 0)),
            pl.BlockSpec((1, R, D), lambda i: (1, i, 0)),
            pl.BlockSpec((2, R, 1), lambda i: (0, i, 0)),
            pl.BlockSpec((D, D), lambda i: (0, 0)),
            pl.BlockSpec((D, D), lambda i: (0, 0)),
            pl.BlockSpec((1, D), lambda i: (0, 0)),
        ],
        out_specs=pl.BlockSpec((R, D), lambda i: (i, 0)),
        out_shape=jax.ShapeDtypeStruct((N, D), jnp.float32),
    )(x, part3, part3, deg3, W_s, W_n, b2d)


def kernel(x, edge_index, W_self1, W_neigh1, b1, W_self2, W_neigh2, b2):
    src = edge_index[0].astype(jnp.int32)
    dst = edge_index[1].astype(jnp.int32)
    z2 = jnp.zeros((NP, D), jnp.float32)
    z1 = jnp.zeros((NP,), jnp.float32)

    part1, degp = _sc_agg(x, src, dst, z2, z1, with_deg=True)
    h = _tc_dense(x, part1, degp, W_self1, W_neigh1, b1, relu=True)
    (part2,) = _sc_agg(h, src, dst, z2, z1, with_deg=False)
    out2 = _tc_dense(h, part2, degp, W_self2, W_neigh2, b2, relu=False)
    src_feat, dst_feat = _sc_gather_out(out2, src, dst)
    return (src_feat, dst_feat)
